# Initial kernel scaffold; baseline (speedup 1.0000x reference)
import jax
import jax.numpy as jnp
from jax.experimental import pallas as pl


def kernel(x, target):
    raise NotImplementedError("write your pallas kernel here")


# speedup vs baseline: 8.2467x; 8.2467x over previous
import math

import jax
import jax.numpy as jnp
from jax.experimental import pallas as pl
from jax.experimental.pallas import tpu as pltpu

_SIZE = 32000
_PAD = 0
_SMOOTH = 0.1
_CONF = 1.0 - _SMOOTH
_EPS = _SMOOTH / (_SIZE - 1)
_C = (_SIZE - 1) * _EPS * math.log(_EPS) + _CONF * math.log(_CONF)

_R = 128
_CB = 16000


def _ls_kernel(tgt_ref, x_ref, out_ref, acc_ref, tok_ref):
    i = pl.program_id(0)
    j = pl.program_id(1)
    ni = pl.num_programs(0)
    nj = pl.num_programs(1)

    @pl.when((i == 0) & (j == 0))
    def _init():
        acc_ref[0, 0] = 0.0
        tok_ref[0, 0] = 0.0

    x = x_ref[...]
    tgt = tgt_ref[0]
    tgt_col = tgt.reshape(_R, 1)
    maskv = tgt_col != _PAD

    rowsum = jnp.sum(x, axis=1, keepdims=True)
    col = jax.lax.broadcasted_iota(jnp.int32, (_R, _CB), 1) + j * _CB
    xt = jnp.sum(jnp.where(col == tgt_col, x, 0.0), axis=1, keepdims=True)

    contrib = jnp.where(maskv, -_EPS * rowsum - (_CONF - _EPS) * xt, 0.0)
    acc_ref[0, 0] += jnp.sum(contrib)

    @pl.when(j == 0)
    def _per_row_once():
        mask_cnt = jnp.sum(maskv.astype(jnp.float32))
        acc_ref[0, 0] += _C * mask_cnt
        tok_ref[0, 0] += mask_cnt

    @pl.when((i == ni - 1) & (j == nj - 1))
    def _finish():
        out_ref[0, 0] = acc_ref[0, 0] / tok_ref[0, 0]


def kernel(x, target):
    n = x.shape[0]
    g = n // _R
    tgt = target.astype(jnp.int32).reshape(g, 1, _R)
    out = pl.pallas_call(
        _ls_kernel,
        grid=(g, _SIZE // _CB),
        in_specs=[
            pl.BlockSpec((1, 1, _R), lambda i, j: (i, 0, 0)),
            pl.BlockSpec((_R, _CB), lambda i, j: (i, j)),
        ],
        out_specs=pl.BlockSpec(memory_space=pltpu.SMEM),
        out_shape=jax.ShapeDtypeStruct((1, 1), jnp.float32),
        scratch_shapes=[
            pltpu.SMEM((1, 1), jnp.float32),
            pltpu.SMEM((1, 1), jnp.float32),
        ],
    )(tgt, x)
    return out[0, 0]


# speedup vs baseline: 8.5861x; 1.0412x over previous
import math

import jax
import jax.numpy as jnp
from jax.experimental import pallas as pl
from jax.experimental.pallas import tpu as pltpu

_SIZE = 32000
_PAD = 0
_SMOOTH = 0.1
_CONF = 1.0 - _SMOOTH
_EPS = _SMOOTH / (_SIZE - 1)
_C = (_SIZE - 1) * _EPS * math.log(_EPS) + _CONF * math.log(_CONF)

_R = 128
_CB = 32000


def _ls_kernel(tgt_ref, x_ref, out_ref, acc_ref, tok_ref):
    i = pl.program_id(0)
    j = pl.program_id(1)
    ni = pl.num_programs(0)
    nj = pl.num_programs(1)

    @pl.when((i == 0) & (j == 0))
    def _init():
        acc_ref[0, 0] = 0.0
        tok_ref[0, 0] = 0.0

    x = x_ref[...]
    tgt = tgt_ref[0]
    tgt_col = tgt.reshape(_R, 1)
    maskv = tgt_col != _PAD

    rowsum = jnp.sum(x, axis=1, keepdims=True)
    col = jax.lax.broadcasted_iota(jnp.int32, (_R, _CB), 1) + j * _CB
    xt = jnp.sum(jnp.where(col == tgt_col, x, 0.0), axis=1, keepdims=True)

    contrib = jnp.where(maskv, -_EPS * rowsum - (_CONF - _EPS) * xt, 0.0)
    acc_ref[0, 0] += jnp.sum(contrib)

    @pl.when(j == 0)
    def _per_row_once():
        mask_cnt = jnp.sum(maskv.astype(jnp.float32))
        acc_ref[0, 0] += _C * mask_cnt
        tok_ref[0, 0] += mask_cnt

    @pl.when((i == ni - 1) & (j == nj - 1))
    def _finish():
        out_ref[0, 0] = acc_ref[0, 0] / tok_ref[0, 0]


def kernel(x, target):
    n = x.shape[0]
    g = n // _R
    tgt = target.astype(jnp.int32).reshape(g, 1, _R)
    out = pl.pallas_call(
        _ls_kernel,
        grid=(g, _SIZE // _CB),
        in_specs=[
            pl.BlockSpec((1, 1, _R), lambda i, j: (i, 0, 0)),
            pl.BlockSpec((_R, _CB), lambda i, j: (i, j)),
        ],
        out_specs=pl.BlockSpec(memory_space=pltpu.SMEM),
        out_shape=jax.ShapeDtypeStruct((1, 1), jnp.float32),
        scratch_shapes=[
            pltpu.SMEM((1, 1), jnp.float32),
            pltpu.SMEM((1, 1), jnp.float32),
        ],
    )(tgt, x)
    return out[0, 0]
